# baseline (speedup 1.0000x reference)
import functools

import jax
import jax.numpy as jnp
from jax import lax
from jax.experimental import pallas as pl
from jax.experimental.pallas import tpu as pltpu
from jax.experimental.pallas import tpu_sc as plsc


def kernel(user, item, user_factors, item_factors):
    B = user.shape[0]
    N_U, D = user_factors.shape
    N_I = item_factors.shape[0]

    info = plsc.get_sparse_core_info()
    NC, NS, L = info.num_cores, info.num_subcores, info.num_lanes
    NW = NC * NS
    b_w = B // NW
    KQ = 8
    NQ = D // KQ

    mesh = plsc.VectorSubcoreMesh(core_axis_name="c", subcore_axis_name="s")

    @functools.partial(
        pl.kernel,
        mesh=mesh,
        out_type=jax.ShapeDtypeStruct((B,), jnp.float32),
        compiler_params=pltpu.CompilerParams(needs_layout_passes=False),
        scratch_types=[
            pltpu.VMEM((b_w,), jnp.int32),
            pltpu.VMEM((b_w,), jnp.int32),
            pltpu.VMEM((b_w,), jnp.int32),
            pltpu.VMEM((b_w,), jnp.int32),
            pltpu.VMEM((2, L, KQ, 128), jnp.float32),
            pltpu.VMEM((2, L, KQ, 128), jnp.float32),
            pltpu.VMEM((b_w,), jnp.float32),
            pltpu.SemaphoreType.DMA,
            pltpu.SemaphoreType.DMA,
        ],
    )
    def sc_kernel(user_hbm, item_hbm, uft_hbm, ift_hbm, out_hbm,
                  uoff, ioff, ucol, icol, u_blk, i_blk, out_v, sem0, sem1):
        sems = [sem0, sem1]
        wid = lax.axis_index("s") * NC + lax.axis_index("c")
        base = wid * b_w

        pltpu.sync_copy(user_hbm.at[pl.ds(base, b_w)], uoff)
        pltpu.sync_copy(item_hbm.at[pl.ds(base, b_w)], ioff)

        for t in range(b_w // L):
            sl = pl.ds(t * L, L)
            v = uoff[sl]
            v = jnp.where(v == 0, N_U - 1, v - 1)
            uoff[sl] = v & ~jnp.int32(127)
            ucol[sl] = v & 127
            w = ioff[sl]
            w = jnp.where(w == 0, N_I - 1, w - 1)
            ioff[sl] = w & ~jnp.int32(127)
            icol[sl] = w & 127

        lanes = lax.iota(jnp.int32, L)

        def fire_quarter(uvec, ivec, q, buf):
            for m in range(L):
                pltpu.make_async_copy(
                    uft_hbm.at[pl.ds(q * KQ, KQ),
                               pl.ds(pl.multiple_of(uvec[m], 128), 128)],
                    u_blk.at[buf, m], sems[buf]).start()
                pltpu.make_async_copy(
                    ift_hbm.at[pl.ds(q * KQ, KQ),
                               pl.ds(pl.multiple_of(ivec[m], 128), 128)],
                    i_blk.at[buf, m], sems[buf]).start()

        def drain_quarter(buf):
            for _ in range(2 * L):
                pltpu.make_async_copy(
                    uft_hbm.at[pl.ds(0, KQ), pl.ds(0, 128)],
                    u_blk.at[buf, 0], sems[buf]).wait()

        def body(g, carry):
            sl = pl.ds(g * L, L)
            uvec = uoff[sl]
            ivec = ioff[sl]
            uc = ucol[sl]
            ic = icol[sl]
            acc = jnp.zeros((L,), jnp.float32)
            fire_quarter(uvec, ivec, 0, 0)
            for q in range(NQ):
                buf = q % 2
                drain_quarter(buf)
                if q + 1 < NQ:
                    fire_quarter(uvec, ivec, q + 1, (q + 1) % 2)
                for k in range(KQ):
                    krow = jnp.full((L,), k, jnp.int32)
                    uk = plsc.load_gather(u_blk.at[buf], [lanes, krow, uc])
                    ik = plsc.load_gather(i_blk.at[buf], [lanes, krow, ic])
                    acc = acc + uk * ik
            out_v[sl] = acc * 5.0
            return carry

        lax.fori_loop(0, b_w // L, body, 0)

        pltpu.sync_copy(out_v, out_hbm.at[pl.ds(base, b_w)])

    return sc_kernel(user, item, user_factors.T, item_factors.T)


# speedup vs baseline: 1.2003x; 1.2003x over previous
import functools

import jax
import jax.numpy as jnp
from jax import lax
from jax.experimental import pallas as pl
from jax.experimental.pallas import tpu as pltpu
from jax.experimental.pallas import tpu_sc as plsc


def _vperm(x, idx):
    dnums = lax.GatherDimensionNumbers(
        offset_dims=(), collapsed_slice_dims=(0,), start_index_map=(0,))
    return lax.gather(x, idx[:, None], dnums, slice_sizes=(1,),
                      mode=lax.GatherScatterMode.PROMISE_IN_BOUNDS)


def kernel(user, item, user_factors, item_factors):
    B = user.shape[0]
    N_U, D = user_factors.shape
    N_I = item_factors.shape[0]

    info = plsc.get_sparse_core_info()
    NC, NS, L = info.num_cores, info.num_subcores, info.num_lanes
    NW = NC * NS
    b_w = B // NW
    SG = L // 2

    mesh = plsc.VectorSubcoreMesh(core_axis_name="c", subcore_axis_name="s")

    @functools.partial(
        pl.kernel,
        mesh=mesh,
        out_type=jax.ShapeDtypeStruct((B,), jnp.float32),
        compiler_params=pltpu.CompilerParams(needs_layout_passes=False),
        scratch_types=[
            pltpu.VMEM((b_w,), jnp.int32),
            pltpu.VMEM((b_w,), jnp.int32),
            pltpu.VMEM((b_w,), jnp.int32),
            pltpu.VMEM((b_w,), jnp.int32),
            pltpu.VMEM((SG, D, 128), jnp.float32),
            pltpu.VMEM((SG, D, 128), jnp.float32),
            pltpu.VMEM((b_w,), jnp.float32),
            pltpu.SemaphoreType.DMA,
        ],
    )
    def sc_kernel(user_hbm, item_hbm, uft_hbm, ift_hbm, out_hbm,
                  uoff, ioff, ucol, icol, u_blk, i_blk, out_v, sem):
        wid = lax.axis_index("s") * NC + lax.axis_index("c")
        base = wid * b_w

        pltpu.sync_copy(user_hbm.at[pl.ds(base, b_w)], uoff)
        pltpu.sync_copy(item_hbm.at[pl.ds(base, b_w)], ioff)

        for t in range(b_w // L):
            sl = pl.ds(t * L, L)
            v = uoff[sl]
            v = jnp.where(v == 0, N_U - 1, v - 1)
            uoff[sl] = v & ~jnp.int32(127)
            ucol[sl] = v & 127
            w = ioff[sl]
            w = jnp.where(w == 0, N_I - 1, w - 1)
            ioff[sl] = w & ~jnp.int32(127)
            icol[sl] = w & 127

        lanes = lax.iota(jnp.int32, L)
        slot = lanes & (SG - 1)
        khalf = lanes >> 3
        fold = lanes ^ SG

        def body(g, carry):
            sl = pl.ds(g * L, L)
            uvec = uoff[sl]
            ivec = ioff[sl]
            uc = ucol[sl]
            ic = icol[sl]
            halves = []
            for hb in range(2):
                copies = []
                for m in range(SG):
                    mm = hb * SG + m
                    copies.append(pltpu.make_async_copy(
                        uft_hbm.at[:, pl.ds(pl.multiple_of(uvec[mm], 128),
                                            128)],
                        u_blk.at[m], sem))
                    copies.append(pltpu.make_async_copy(
                        ift_hbm.at[:, pl.ds(pl.multiple_of(ivec[mm], 128),
                                            128)],
                        i_blk.at[m], sem))
                for cp in copies:
                    cp.start()
                for cp in copies:
                    cp.wait()
                ucs = _vperm(uc, slot + hb * SG)
                ics = _vperm(ic, slot + hb * SG)
                acc = jnp.zeros((L,), jnp.float32)
                for kp in range(D // 2):
                    krow = kp * 2 + khalf
                    uk = plsc.load_gather(u_blk, [slot, krow, ucs])
                    ik = plsc.load_gather(i_blk, [slot, krow, ics])
                    acc = acc + uk * ik
                halves.append(
                    acc + _vperm(acc, fold))
            out_v[sl] = jnp.where(khalf == 0, halves[0], halves[1]) * 5.0
            return carry

        lax.fori_loop(0, b_w // L, body, 0)

        pltpu.sync_copy(out_v, out_hbm.at[pl.ds(base, b_w)])

    return sc_kernel(user, item, user_factors.T, item_factors.T)
